# Initial kernel scaffold; baseline (speedup 1.0000x reference)
import functools

import jax
import jax.numpy as jnp
from jax import lax
from jax.experimental import pallas as pl
from jax.experimental.pallas import tpu as pltpu
from jax.experimental.pallas import tpu_sc as plsc

USER_N = 10000
ITEM_N = 40000
N_NODES = USER_N + ITEM_N
D = 64
E = 800000
NLAYERS = 3
LMBD = 0.0001
BATCH = 4096

NC = 2
NS = 16
H = 25088
NPAD = NC * H
RPT = H // NS
GR = H
ACC_ROWS = H + 8
EPT = E // NS
K = 400
NCHUNK = EPT // K
RNORM = 224
BW = BATCH // (NC * NS)

_f32 = jnp.float32
_i32 = jnp.int32


def _rsqrt16(s):
    i = plsc.bitcast(s, _i32)
    y = plsc.bitcast(jnp.int32(0x5F3759DF) - (i >> 1), _f32)
    for _ in range(3):
        y = y * (1.5 - 0.5 * s * y * y)
    return y


def _iota16():
    return lax.iota(_i32, 16)


def _layer_body(src_hbm, dst_hbm, w_hbm, h_hbm, out_hbm,
                src_v, dst_v, w_v, loc_v, rows_v, nst_v, acc_sh, sem):
    c = lax.axis_index("c")
    s = lax.axis_index("s")
    lo = c * H

    z = jnp.zeros((16,), _f32)

    def _zrow(r, _):
        for j in range(D // 16):
            nst_v[r, pl.ds(j * 16, 16)] = z
        return 0

    lax.fori_loop(0, RNORM, _zrow, 0)
    for r in range(RPT // RNORM):
        pltpu.sync_copy(nst_v, acc_sh.at[pl.ds(s * RPT + r * RNORM, RNORM)])

    @pl.when(s == 0)
    def _():
        pltpu.sync_copy(nst_v.at[pl.ds(0, 8)], acc_sh.at[pl.ds(H, 8)])

    plsc.subcore_barrier()

    def _chunk(i, _):
        base = s * EPT + i * K
        pltpu.sync_copy(src_hbm.at[pl.ds(base, K)], src_v)
        pltpu.sync_copy(dst_hbm.at[pl.ds(base, K)], dst_v)
        pltpu.sync_copy(w_hbm.at[pl.ds(base, K)], w_v)

        def _locs(j, _):
            d = dst_v[pl.ds(j * 16, 16)]
            dl = d - lo
            keep = (dl >= 0) & (dl < H)
            loc_v[pl.ds(j * 16, 16)] = jnp.where(keep, dl, GR)
            return 0

        lax.fori_loop(0, K // 16, _locs, 0)
        pltpu.async_copy(h_hbm.at[src_v], rows_v, sem).wait()

        def _scale(e, _):
            we = w_v[e]
            for j in range(D // 16):
                sl = pl.ds(j * 16, 16)
                rows_v[e, sl] = rows_v[e, sl] * we
            return 0

        lax.fori_loop(0, K, _scale, 0)
        pltpu.sync_copy(rows_v, acc_sh.at[loc_v], add=True)
        return 0

    lax.fori_loop(0, NCHUNK, _chunk, 0)
    plsc.subcore_barrier()

    ii = _iota16()
    for r in range(RPT // RNORM):
        lrow = s * RPT + r * RNORM
        pltpu.sync_copy(acc_sh.at[pl.ds(lrow, RNORM)], nst_v)

        def _grp(g, _):
            ridx = g * 16 + ii
            ss = jnp.zeros((16,), _f32)
            cols = []
            for j in range(D):
                v = plsc.load_gather(nst_v, [ridx, jnp.full((16,), j, _i32)])
                cols.append(v)
                ss = ss + v * v
            inv = _rsqrt16(ss)
            for j in range(D):
                plsc.store_scatter(
                    nst_v, [ridx, jnp.full((16,), j, _i32)], cols[j] * inv)
            return 0

        lax.fori_loop(0, RNORM // 16, _grp, 0)
        pltpu.sync_copy(nst_v, out_hbm.at[pl.ds(c * H + lrow, RNORM)])


def _make_layer():
    mesh = plsc.VectorSubcoreMesh(core_axis_name="c", subcore_axis_name="s")
    return pl.kernel(
        _layer_body,
        out_type=jax.ShapeDtypeStruct((NPAD, D), _f32),
        mesh=mesh,
        scratch_types=[
            pltpu.VMEM((K,), _i32),
            pltpu.VMEM((K,), _i32),
            pltpu.VMEM((K,), _f32),
            pltpu.VMEM((K,), _i32),
            pltpu.VMEM((K, D), _f32),
            pltpu.VMEM((RNORM, D), _f32),
            pltpu.VMEM_SHARED((ACC_ROWS, D), _f32),
            pltpu.SemaphoreType.DMA,
        ],
    )


def _final_body(h0, h1, h2, h3, uid_hbm, pid_hbm, nid_hbm,
                pos_hbm, neg_hbm, reg_hbm,
                idx_v, u_v, p_v, n_v, pos_v, neg_v, reg_v, sem):
    c = lax.axis_index("c")
    s = lax.axis_index("s")
    wid = s * NC + c
    base = wid * BW
    ii = _iota16()

    def _sumrows(id_hbm, buf):
        pltpu.sync_copy(id_hbm.at[pl.ds(base, BW)], idx_v)
        pltpu.async_copy(h0.at[idx_v], buf, sem).wait()
        for t in (h1, h2, h3):
            pltpu.async_copy(t.at[idx_v], buf, sem, add=True).wait()

    _sumrows(uid_hbm, u_v)
    _sumrows(pid_hbm, p_v)
    _sumrows(nid_hbm, n_v)

    racc0 = jnp.zeros((16,), _f32)

    def _grp(g, racc):
        ridx = g * 16 + ii
        pacc = jnp.zeros((16,), _f32)
        nacc = jnp.zeros((16,), _f32)
        for j in range(D):
            cj = jnp.full((16,), j, _i32)
            gu = plsc.load_gather(u_v, [ridx, cj])
            gp = plsc.load_gather(p_v, [ridx, cj])
            gn = plsc.load_gather(n_v, [ridx, cj])
            pacc = pacc + gu * gp
            nacc = nacc + gu * gn
            racc = racc + gu * gu + gp * gp + gn * gn
        pos_v[pl.ds(g * 16, 16)] = pacc * 0.0625
        neg_v[pl.ds(g * 16, 16)] = nacc * 0.0625
        return racc

    racc = lax.fori_loop(0, BW // 16, _grp, racc0)
    reg_v[pl.ds(0, 16)] = racc * 0.0625

    pltpu.sync_copy(pos_v, pos_hbm.at[pl.ds(base, BW)])
    pltpu.sync_copy(neg_v, neg_hbm.at[pl.ds(base, BW)])
    pltpu.sync_copy(reg_v, reg_hbm.at[wid])


def _make_final():
    mesh = plsc.VectorSubcoreMesh(core_axis_name="c", subcore_axis_name="s")
    return pl.kernel(
        _final_body,
        out_type=(
            jax.ShapeDtypeStruct((BATCH,), _f32),
            jax.ShapeDtypeStruct((BATCH,), _f32),
            jax.ShapeDtypeStruct((NC * NS, 16), _f32),
        ),
        mesh=mesh,
        scratch_types=[
            pltpu.VMEM((BW,), _i32),
            pltpu.VMEM((BW, D), _f32),
            pltpu.VMEM((BW, D), _f32),
            pltpu.VMEM((BW, D), _f32),
            pltpu.VMEM((BW,), _f32),
            pltpu.VMEM((BW,), _f32),
            pltpu.VMEM((16,), _f32),
            pltpu.SemaphoreType.DMA,
        ],
    )


def _loss_body(pos_ref, neg_ref, reg_ref, out_ref):
    z = pos_ref[...] - neg_ref[...]
    loss = -jnp.mean(jnp.log(jax.nn.sigmoid(z)))
    reg = jnp.sum(reg_ref[...])
    out_ref[...] = jnp.full((1, 1), loss + LMBD * (reg * 0.5) / BATCH, _f32)


def kernel(edge_index, edge_weight, user_table, item_table,
           user_id, item_id, neg_item_id):
    src = edge_index[0].astype(_i32)
    dst = edge_index[1].astype(_i32)
    w = edge_weight.astype(_f32)
    pad = jnp.zeros((NPAD - N_NODES, D), _f32)
    h0 = jnp.concatenate([user_table, item_table, pad], axis=0)

    layer = _make_layer()
    h1 = layer(src, dst, w, h0)
    h2 = layer(src, dst, w, h1)
    h3 = layer(src, dst, w, h2)

    final = _make_final()
    pos, neg, regp = final(
        h0, h1, h2, h3,
        user_id.astype(_i32),
        (item_id + USER_N).astype(_i32),
        (neg_item_id + USER_N).astype(_i32),
    )

    out = pl.pallas_call(
        _loss_body,
        out_shape=jax.ShapeDtypeStruct((1, 1), _f32),
    )(pos.reshape(32, 128), neg.reshape(32, 128), regp)
    return out[0, 0]


# speedup vs baseline: 2.6840x; 2.6840x over previous
import functools

import jax
import jax.numpy as jnp
from jax import lax
from jax.experimental import pallas as pl
from jax.experimental.pallas import tpu as pltpu
from jax.experimental.pallas import tpu_sc as plsc

USER_N = 10000
ITEM_N = 40000
N_NODES = USER_N + ITEM_N
D = 64
E = 800000
NLAYERS = 3
LMBD = 0.0001
BATCH = 4096

NC = 2
NS = 16
H = 25088
NPAD = NC * H
RPT = H // NS
GR = H
ACC_ROWS = H + 8
EPT = E // NS
K = 400
NCHUNK = EPT // K
RNORM = 56
BW = BATCH // (NC * NS)

_f32 = jnp.float32
_i32 = jnp.int32


def _rsqrt16(s):
    i = lax.bitcast_convert_type(s, _i32)
    y = lax.bitcast_convert_type(jnp.int32(0x5F3759DF) - (i >> 1), _f32)
    for _ in range(3):
        y = y * (1.5 - 0.5 * s * y * y)
    return y


def _iota16():
    return lax.iota(_i32, 16)


def _hsum16(v):
    ii = _iota16()
    for k in (8, 4, 2, 1):
        v = v + v[ii ^ k]
    return v


def _layer_body(src_hbm, dst_hbm, w_hbm, h_hbm, out_hbm,
                src_v, dst_v, w_v, rows_v, nst_v, acc_sh, sem):
    c = lax.axis_index("c")
    s = lax.axis_index("s")
    lo = c * H

    z = jnp.zeros((16,), _f32)

    def _zrow(r, _):
        for j in range(D // 16):
            nst_v[r, pl.ds(j * 16, 16)] = z
        return 0

    lax.fori_loop(0, RNORM, _zrow, 0)
    for r in range(RPT // RNORM):
        pltpu.sync_copy(nst_v, acc_sh.at[pl.ds(s * RPT + r * RNORM, RNORM)])

    @pl.when(s == 0)
    def _():
        pltpu.sync_copy(nst_v.at[pl.ds(0, 8)], acc_sh.at[pl.ds(H, 8)])

    plsc.subcore_barrier()

    def _chunk(i, _):
        base = s * EPT + i * K
        pltpu.sync_copy(src_hbm.at[pl.ds(base, K)], src_v)
        pltpu.sync_copy(dst_hbm.at[pl.ds(base, K)], dst_v)
        pltpu.sync_copy(w_hbm.at[pl.ds(base, K)], w_v)

        def _locs(j, _):
            sl = pl.ds(j * 16, 16)
            dl = dst_v[sl] - lo
            keep = (dl >= 0) & (dl < H)
            dst_v[sl] = jnp.where(keep, dl, GR)
            return 0

        lax.fori_loop(0, K // 16, _locs, 0)
        pltpu.async_copy(h_hbm.at[src_v], rows_v, sem).wait()

        def _scale(g, _):
            wv = w_v[pl.ds(g * 16, 16)]
            for m in range(16):
                e = g * 16 + m
                we = wv[m]
                for j in range(D // 16):
                    sl = pl.ds(j * 16, 16)
                    rows_v[e, sl] = rows_v[e, sl] * we
            return 0

        lax.fori_loop(0, K // 16, _scale, 0)
        pltpu.sync_copy(rows_v, acc_sh.at[dst_v], add=True)
        return 0

    lax.fori_loop(0, NCHUNK, _chunk, 0)
    plsc.subcore_barrier()

    for r in range(RPT // RNORM):
        lrow = s * RPT + r * RNORM
        pltpu.sync_copy(acc_sh.at[pl.ds(lrow, RNORM)], nst_v)

        def _row(e, _):
            vs = [nst_v[e, pl.ds(j * 16, 16)] for j in range(D // 16)]
            ss = vs[0] * vs[0]
            for v in vs[1:]:
                ss = ss + v * v
            inv = _rsqrt16(_hsum16(ss))
            for j in range(D // 16):
                nst_v[e, pl.ds(j * 16, 16)] = vs[j] * inv
            return 0

        lax.fori_loop(0, RNORM, _row, 0)
        pltpu.sync_copy(nst_v, out_hbm.at[pl.ds(c * H + lrow, RNORM)])


def _make_layer():
    mesh = plsc.VectorSubcoreMesh(core_axis_name="c", subcore_axis_name="s")
    return pl.kernel(
        _layer_body,
        out_type=jax.ShapeDtypeStruct((NPAD, D), _f32),
        mesh=mesh,
        compiler_params=pltpu.CompilerParams(use_tc_tiling_on_sc=False),
        scratch_types=[
            pltpu.VMEM((K,), _i32),
            pltpu.VMEM((K,), _i32),
            pltpu.VMEM((K,), _f32),
            pltpu.VMEM((K, D), _f32),
            pltpu.VMEM((RNORM, D), _f32),
            pltpu.VMEM_SHARED((ACC_ROWS, D), _f32),
            pltpu.SemaphoreType.DMA,
        ],
    )


def _final_body(h0, h1, h2, h3, uid_hbm, pid_hbm, nid_hbm,
                pos_hbm, neg_hbm, reg_hbm,
                idx_v, u_v, p_v, n_v, pos_v, neg_v, reg_v, sem):
    c = lax.axis_index("c")
    s = lax.axis_index("s")
    wid = s * NC + c
    base = wid * BW
    ii = _iota16()

    def _sumrows(id_hbm, buf):
        pltpu.sync_copy(id_hbm.at[pl.ds(base, BW)], idx_v)
        pltpu.async_copy(h0.at[idx_v], buf, sem).wait()
        for t in (h1, h2, h3):
            pltpu.async_copy(t.at[idx_v], buf, sem, add=True).wait()

    _sumrows(uid_hbm, u_v)
    _sumrows(pid_hbm, p_v)
    _sumrows(nid_hbm, n_v)

    racc0 = jnp.zeros((16,), _f32)

    def _grp(g, racc):
        pos16 = jnp.zeros((16,), _f32)
        neg16 = jnp.zeros((16,), _f32)
        for m in range(16):
            e = g * 16 + m
            pacc = jnp.zeros((16,), _f32)
            nacc = jnp.zeros((16,), _f32)
            for j in range(D // 16):
                sl = pl.ds(j * 16, 16)
                gu = u_v[e, sl]
                gp = p_v[e, sl]
                gn = n_v[e, sl]
                pacc = pacc + gu * gp
                nacc = nacc + gu * gn
                racc = racc + gu * gu + gp * gp + gn * gn
            lane = ii == m
            pos16 = jnp.where(lane, _hsum16(pacc), pos16)
            neg16 = jnp.where(lane, _hsum16(nacc), neg16)
        pos_v[pl.ds(g * 16, 16)] = pos16 * 0.0625
        neg_v[pl.ds(g * 16, 16)] = neg16 * 0.0625
        return racc

    racc = lax.fori_loop(0, BW // 16, _grp, racc0)
    reg_v[pl.ds(0, 16)] = racc * 0.0625

    pltpu.sync_copy(pos_v, pos_hbm.at[pl.ds(base, BW)])
    pltpu.sync_copy(neg_v, neg_hbm.at[pl.ds(base, BW)])
    pltpu.sync_copy(reg_v, reg_hbm.at[wid])


def _make_final():
    mesh = plsc.VectorSubcoreMesh(core_axis_name="c", subcore_axis_name="s")
    return pl.kernel(
        _final_body,
        out_type=(
            jax.ShapeDtypeStruct((BATCH,), _f32),
            jax.ShapeDtypeStruct((BATCH,), _f32),
            jax.ShapeDtypeStruct((NC * NS, 16), _f32),
        ),
        mesh=mesh,
        compiler_params=pltpu.CompilerParams(use_tc_tiling_on_sc=False),
        scratch_types=[
            pltpu.VMEM((BW,), _i32),
            pltpu.VMEM((BW, D), _f32),
            pltpu.VMEM((BW, D), _f32),
            pltpu.VMEM((BW, D), _f32),
            pltpu.VMEM((BW,), _f32),
            pltpu.VMEM((BW,), _f32),
            pltpu.VMEM((16,), _f32),
            pltpu.SemaphoreType.DMA,
        ],
    )


def _loss_body(pos_ref, neg_ref, reg_ref, out_ref):
    z = pos_ref[...] - neg_ref[...]
    loss = -jnp.mean(jnp.log(jax.nn.sigmoid(z)))
    reg = jnp.sum(reg_ref[...])
    out_ref[...] = jnp.full((1, 1), loss + LMBD * (reg * 0.5) / BATCH, _f32)


def kernel(edge_index, edge_weight, user_table, item_table,
           user_id, item_id, neg_item_id):
    src = edge_index[0].astype(_i32)
    dst = edge_index[1].astype(_i32)
    w = edge_weight.astype(_f32)
    pad = jnp.zeros((NPAD - N_NODES, D), _f32)
    h0 = jnp.concatenate([user_table, item_table, pad], axis=0)

    layer = _make_layer()
    h1 = layer(src, dst, w, h0)
    h2 = layer(src, dst, w, h1)
    h3 = layer(src, dst, w, h2)

    final = _make_final()
    pos, neg, regp = final(
        h0, h1, h2, h3,
        user_id.astype(_i32),
        (item_id + USER_N).astype(_i32),
        (neg_item_id + USER_N).astype(_i32),
    )

    out = pl.pallas_call(
        _loss_body,
        out_shape=jax.ShapeDtypeStruct((1, 1), _f32),
    )(pos.reshape(32, 128), neg.reshape(32, 128), regp)
    return out[0, 0]


# speedup vs baseline: 5.6764x; 2.1149x over previous
import functools

import jax
import jax.numpy as jnp
from jax import lax
from jax.experimental import pallas as pl
from jax.experimental.pallas import tpu as pltpu
from jax.experimental.pallas import tpu_sc as plsc

USER_N = 10000
ITEM_N = 40000
N_NODES = USER_N + ITEM_N
D = 64
HD = D // 2
E = 800000
LMBD = 0.0001
BATCH = 4096

NC = 2
NS = 16
NPAD = 50176
RPT = NPAD // NS
EPT = E // NS
K = 400
NCHUNK = EPT // K
BW = BATCH // (NC * NS)
NROWBLK = 512
EPS = 1e-12

_f32 = jnp.float32
_i32 = jnp.int32


def _iota16():
    return lax.iota(_i32, 16)


def _hsum16(v):
    ii = _iota16()
    for k in (8, 4, 2, 1):
        v = v + v[ii ^ k]
    return v


def _accum_body(src_hbm, dst_hbm, w_hbm, h_hbm, out_hbm,
                src_v, dst_v, w_v, rows_v, acc_sh, sem):
    c = lax.axis_index("c")
    s = lax.axis_index("s")

    z = jnp.zeros((16,), _f32)

    def _zrow(r, _):
        for j in range(HD // 16):
            rows_v[r, pl.ds(j * 16, 16)] = z
        return 0

    lax.fori_loop(0, K, _zrow, 0)
    for r in range(RPT // K):
        pltpu.sync_copy(rows_v, acc_sh.at[pl.ds(s * RPT + r * K, K)])
    rem = RPT - (RPT // K) * K
    if rem:
        pltpu.sync_copy(rows_v.at[pl.ds(0, rem)],
                        acc_sh.at[pl.ds(s * RPT + (RPT // K) * K, rem)])
    plsc.subcore_barrier()

    def _chunk(i, _):
        base = s * EPT + i * K
        pltpu.sync_copy(src_hbm.at[pl.ds(base, K)], src_v)
        pltpu.sync_copy(dst_hbm.at[pl.ds(base, K)], dst_v)
        pltpu.sync_copy(w_hbm.at[pl.ds(base, K)], w_v)
        pltpu.async_copy(h_hbm.at[c].at[src_v], rows_v, sem).wait()

        def _scale(g, _):
            wv = w_v[pl.ds(g * 16, 16)]
            for m in range(16):
                e = g * 16 + m
                we = wv[m]
                for j in range(HD // 16):
                    sl = pl.ds(j * 16, 16)
                    rows_v[e, sl] = rows_v[e, sl] * we
            return 0

        lax.fori_loop(0, K // 16, _scale, 0)
        pltpu.sync_copy(rows_v, acc_sh.at[dst_v], add=True)
        return 0

    lax.fori_loop(0, NCHUNK, _chunk, 0)
    plsc.subcore_barrier()

    for r in range(RPT // K):
        sl = pl.ds(s * RPT + r * K, K)
        pltpu.sync_copy(acc_sh.at[sl], out_hbm.at[c].at[sl])
    if rem:
        sl = pl.ds(s * RPT + (RPT // K) * K, rem)
        pltpu.sync_copy(acc_sh.at[sl], out_hbm.at[c].at[sl])


def _make_accum():
    mesh = plsc.VectorSubcoreMesh(core_axis_name="c", subcore_axis_name="s")
    return pl.kernel(
        _accum_body,
        out_type=jax.ShapeDtypeStruct((NC, NPAD, HD), _f32),
        mesh=mesh,
        compiler_params=pltpu.CompilerParams(use_tc_tiling_on_sc=False),
        scratch_types=[
            pltpu.VMEM((K,), _i32),
            pltpu.VMEM((K,), _i32),
            pltpu.VMEM((K,), _f32),
            pltpu.VMEM((K, HD), _f32),
            pltpu.VMEM_SHARED((NPAD, HD), _f32),
            pltpu.SemaphoreType.DMA,
        ],
    )


def _norm_body(agg_ref, out_ref):
    x = agg_ref[...]
    ss = jnp.sum(x * x, axis=(0, 2))
    inv = 1.0 / jnp.maximum(jnp.sqrt(ss), EPS)
    out_ref[...] = x * inv[None, :, None]


def _normalize(agg):
    return pl.pallas_call(
        _norm_body,
        grid=(NPAD // NROWBLK,),
        in_specs=[pl.BlockSpec((NC, NROWBLK, HD), lambda i: (0, i, 0))],
        out_specs=pl.BlockSpec((NC, NROWBLK, HD), lambda i: (0, i, 0)),
        out_shape=jax.ShapeDtypeStruct((NC, NPAD, HD), _f32),
    )(agg)


def _final_body(h0, h1, h2, h3, uid_hbm, pid_hbm, nid_hbm,
                pos_hbm, neg_hbm, reg_hbm,
                idx_v, ua_v, ub_v, pa_v, pb_v, na_v, nb_v,
                pos_v, neg_v, reg_v, sem):
    c = lax.axis_index("c")
    s = lax.axis_index("s")
    wid = s * NC + c
    base = wid * BW
    ii = _iota16()

    def _sumrows(id_hbm, bufa, bufb):
        pltpu.sync_copy(id_hbm.at[pl.ds(base, BW)], idx_v)
        pltpu.async_copy(h0.at[0].at[idx_v], bufa, sem).wait()
        pltpu.async_copy(h0.at[1].at[idx_v], bufb, sem).wait()
        for t in (h1, h2, h3):
            pltpu.async_copy(t.at[0].at[idx_v], bufa, sem, add=True).wait()
            pltpu.async_copy(t.at[1].at[idx_v], bufb, sem, add=True).wait()

    _sumrows(uid_hbm, ua_v, ub_v)
    _sumrows(pid_hbm, pa_v, pb_v)
    _sumrows(nid_hbm, na_v, nb_v)

    racc0 = jnp.zeros((16,), _f32)

    def _grp(g, racc):
        pos16 = jnp.zeros((16,), _f32)
        neg16 = jnp.zeros((16,), _f32)
        for m in range(16):
            e = g * 16 + m
            pacc = jnp.zeros((16,), _f32)
            nacc = jnp.zeros((16,), _f32)
            for (ub, pb, nb) in ((ua_v, pa_v, na_v), (ub_v, pb_v, nb_v)):
                for j in range(HD // 16):
                    sl = pl.ds(j * 16, 16)
                    gu = ub[e, sl]
                    gp = pb[e, sl]
                    gn = nb[e, sl]
                    pacc = pacc + gu * gp
                    nacc = nacc + gu * gn
                    racc = racc + gu * gu + gp * gp + gn * gn
            lane = ii == m
            pos16 = jnp.where(lane, _hsum16(pacc), pos16)
            neg16 = jnp.where(lane, _hsum16(nacc), neg16)
        pos_v[pl.ds(g * 16, 16)] = pos16 * 0.0625
        neg_v[pl.ds(g * 16, 16)] = neg16 * 0.0625
        return racc

    racc = lax.fori_loop(0, BW // 16, _grp, racc0)
    reg_v[pl.ds(0, 16)] = racc * 0.0625

    pltpu.sync_copy(pos_v, pos_hbm.at[pl.ds(base, BW)])
    pltpu.sync_copy(neg_v, neg_hbm.at[pl.ds(base, BW)])
    pltpu.sync_copy(reg_v, reg_hbm.at[wid])


def _make_final():
    mesh = plsc.VectorSubcoreMesh(core_axis_name="c", subcore_axis_name="s")
    return pl.kernel(
        _final_body,
        out_type=(
            jax.ShapeDtypeStruct((BATCH,), _f32),
            jax.ShapeDtypeStruct((BATCH,), _f32),
            jax.ShapeDtypeStruct((NC * NS, 16), _f32),
        ),
        mesh=mesh,
        compiler_params=pltpu.CompilerParams(use_tc_tiling_on_sc=False),
        scratch_types=[
            pltpu.VMEM((BW,), _i32),
            pltpu.VMEM((BW, HD), _f32),
            pltpu.VMEM((BW, HD), _f32),
            pltpu.VMEM((BW, HD), _f32),
            pltpu.VMEM((BW, HD), _f32),
            pltpu.VMEM((BW, HD), _f32),
            pltpu.VMEM((BW, HD), _f32),
            pltpu.VMEM((BW,), _f32),
            pltpu.VMEM((BW,), _f32),
            pltpu.VMEM((16,), _f32),
            pltpu.SemaphoreType.DMA,
        ],
    )


def _loss_body(pos_ref, neg_ref, reg_ref, out_ref):
    z = pos_ref[...] - neg_ref[...]
    loss = -jnp.mean(jnp.log(jax.nn.sigmoid(z)))
    reg = jnp.sum(reg_ref[...])
    out_ref[...] = jnp.full((1, 1), loss + LMBD * (reg * 0.5) / BATCH, _f32)


def kernel(edge_index, edge_weight, user_table, item_table,
           user_id, item_id, neg_item_id):
    src = edge_index[0].astype(_i32)
    dst = edge_index[1].astype(_i32)
    w = edge_weight.astype(_f32)
    pad = jnp.zeros((NPAD - N_NODES, D), _f32)
    h0full = jnp.concatenate([user_table, item_table, pad], axis=0)
    h0 = jnp.stack([h0full[:, :HD], h0full[:, HD:]])

    accum = _make_accum()
    h1 = _normalize(accum(src, dst, w, h0))
    h2 = _normalize(accum(src, dst, w, h1))
    h3 = _normalize(accum(src, dst, w, h2))

    final = _make_final()
    pos, neg, regp = final(
        h0, h1, h2, h3,
        user_id.astype(_i32),
        (item_id + USER_N).astype(_i32),
        (neg_item_id + USER_N).astype(_i32),
    )

    out = pl.pallas_call(
        _loss_body,
        out_shape=jax.ShapeDtypeStruct((1, 1), _f32),
    )(pos.reshape(32, 128), neg.reshape(32, 128), regp)
    return out[0, 0]


# speedup vs baseline: 8.9607x; 1.5786x over previous
import functools

import jax
import jax.numpy as jnp
from jax import lax
from jax.experimental import pallas as pl
from jax.experimental.pallas import tpu as pltpu
from jax.experimental.pallas import tpu_sc as plsc

USER_N = 10000
ITEM_N = 40000
N_NODES = USER_N + ITEM_N
D = 64
HD = D // 2
E = 800000
LMBD = 0.0001
BATCH = 4096

NC = 2
NS = 16
NPAD = 50176
RPT = NPAD // NS
EPT = E // NS
K = 400
NCHUNK = EPT // K
BW = BATCH // (NC * NS)
NROWBLK = 512
EPS = 1e-12

_f32 = jnp.float32
_i32 = jnp.int32


def _iota16():
    return lax.iota(_i32, 16)


def _hsum16(v):
    ii = _iota16()
    for k in (8, 4, 2, 1):
        v = v + v[ii ^ k]
    return v


def _accum_body(src_hbm, dst_hbm, w_hbm, h_hbm, out_hbm,
                src0, dst0, w0, src1, dst1, w1, rows0, rows1,
                acc_sh, si0, si1, sg0, sg1):
    c = lax.axis_index("c")
    s = lax.axis_index("s")

    z = jnp.zeros((16,), _f32)

    def _zrow(r, _):
        for j in range(HD // 16):
            rows0[r, pl.ds(j * 16, 16)] = z
        return 0

    lax.fori_loop(0, K, _zrow, 0)
    for r in range(RPT // K):
        pltpu.sync_copy(rows0, acc_sh.at[pl.ds(s * RPT + r * K, K)])
    rem = RPT - (RPT // K) * K
    if rem:
        pltpu.sync_copy(rows0.at[pl.ds(0, rem)],
                        acc_sh.at[pl.ds(s * RPT + (RPT // K) * K, rem)])
    plsc.subcore_barrier()

    sets = ((src0, dst0, w0, rows0, si0, sg0),
            (src1, dst1, w1, rows1, si1, sg1))

    def _start_idx(i, st):
        sv, dv, wv, _, si, _ = st
        base = s * EPT + i * K
        pltpu.async_copy(src_hbm.at[pl.ds(base, K)], sv, si)
        pltpu.async_copy(dst_hbm.at[pl.ds(base, K)], dv, si)
        pltpu.async_copy(w_hbm.at[pl.ds(base, K)], wv, si)

    def _wait_idx(st):
        sv, dv, wv, _, si, _ = st
        pltpu.make_async_copy(src_hbm.at[pl.ds(0, K)], sv, si).wait()
        pltpu.make_async_copy(dst_hbm.at[pl.ds(0, K)], dv, si).wait()
        pltpu.make_async_copy(w_hbm.at[pl.ds(0, K)], wv, si).wait()

    def _start_gather(st):
        sv, _, _, rv, _, sg = st
        pltpu.async_copy(h_hbm.at[c].at[sv], rv, sg)

    def _wait_gather(st):
        sv, _, _, rv, _, sg = st
        pltpu.make_async_copy(h_hbm.at[c].at[sv], rv, sg).wait()

    def _process(st):
        _, dv, wv, rv, _, _ = st

        def _scale(g, _):
            wgrp = wv[pl.ds(g * 16, 16)]
            for m in range(16):
                e = g * 16 + m
                we = wgrp[m]
                for j in range(HD // 16):
                    sl = pl.ds(j * 16, 16)
                    rv[e, sl] = rv[e, sl] * we
            return 0

        lax.fori_loop(0, K // 16, _scale, 0)
        pltpu.sync_copy(rv, acc_sh.at[dv], add=True)

    _start_idx(0, sets[0])
    _wait_idx(sets[0])
    _start_gather(sets[0])
    _start_idx(1, sets[1])

    def _pair(p, _):
        _wait_idx(sets[1])
        _wait_gather(sets[0])
        _start_gather(sets[1])
        _process(sets[0])
        _start_idx(2 * p + 2, sets[0])
        _wait_idx(sets[0])
        _wait_gather(sets[1])
        _start_gather(sets[0])
        _process(sets[1])

        @pl.when(p < (NCHUNK - 1) // 2 - 1)
        def _():
            _start_idx(2 * p + 3, sets[1])

        return 0

    lax.fori_loop(0, (NCHUNK - 1) // 2, _pair, 0)
    _wait_gather(sets[0])
    _process(sets[0])
    plsc.subcore_barrier()

    for r in range(RPT // K):
        sl = pl.ds(s * RPT + r * K, K)
        pltpu.sync_copy(acc_sh.at[sl], out_hbm.at[c].at[sl])
    if rem:
        sl = pl.ds(s * RPT + (RPT // K) * K, rem)
        pltpu.sync_copy(acc_sh.at[sl], out_hbm.at[c].at[sl])


def _make_accum():
    mesh = plsc.VectorSubcoreMesh(core_axis_name="c", subcore_axis_name="s")
    return pl.kernel(
        _accum_body,
        out_type=jax.ShapeDtypeStruct((NC, NPAD, HD), _f32),
        mesh=mesh,
        compiler_params=pltpu.CompilerParams(use_tc_tiling_on_sc=False),
        scratch_types=[
            pltpu.VMEM((K,), _i32),
            pltpu.VMEM((K,), _i32),
            pltpu.VMEM((K,), _f32),
            pltpu.VMEM((K,), _i32),
            pltpu.VMEM((K,), _i32),
            pltpu.VMEM((K,), _f32),
            pltpu.VMEM((K, HD), _f32),
            pltpu.VMEM((K, HD), _f32),
            pltpu.VMEM_SHARED((NPAD, HD), _f32),
            pltpu.SemaphoreType.DMA,
            pltpu.SemaphoreType.DMA,
            pltpu.SemaphoreType.DMA,
            pltpu.SemaphoreType.DMA,
        ],
    )


def _norm_body(agg_ref, out_ref):
    x = agg_ref[...]
    ss = jnp.sum(x * x, axis=(0, 2))
    inv = 1.0 / jnp.maximum(jnp.sqrt(ss), EPS)
    out_ref[...] = x * inv[None, :, None]


def _normalize(agg):
    return pl.pallas_call(
        _norm_body,
        grid=(NPAD // NROWBLK,),
        in_specs=[pl.BlockSpec((NC, NROWBLK, HD), lambda i: (0, i, 0))],
        out_specs=pl.BlockSpec((NC, NROWBLK, HD), lambda i: (0, i, 0)),
        out_shape=jax.ShapeDtypeStruct((NC, NPAD, HD), _f32),
    )(agg)


def _final_body(h0, h1, h2, h3, uid_hbm, pid_hbm, nid_hbm,
                pos_hbm, neg_hbm, reg_hbm,
                idx_v, ua_v, ub_v, pa_v, pb_v, na_v, nb_v,
                pos_v, neg_v, reg_v, sem):
    c = lax.axis_index("c")
    s = lax.axis_index("s")
    wid = s * NC + c
    base = wid * BW
    ii = _iota16()

    def _sumrows(id_hbm, bufa, bufb):
        pltpu.sync_copy(id_hbm.at[pl.ds(base, BW)], idx_v)
        pltpu.async_copy(h0.at[0].at[idx_v], bufa, sem).wait()
        pltpu.async_copy(h0.at[1].at[idx_v], bufb, sem).wait()
        for t in (h1, h2, h3):
            pltpu.async_copy(t.at[0].at[idx_v], bufa, sem, add=True).wait()
            pltpu.async_copy(t.at[1].at[idx_v], bufb, sem, add=True).wait()

    _sumrows(uid_hbm, ua_v, ub_v)
    _sumrows(pid_hbm, pa_v, pb_v)
    _sumrows(nid_hbm, na_v, nb_v)

    racc0 = jnp.zeros((16,), _f32)

    def _grp(g, racc):
        pos16 = jnp.zeros((16,), _f32)
        neg16 = jnp.zeros((16,), _f32)
        for m in range(16):
            e = g * 16 + m
            pacc = jnp.zeros((16,), _f32)
            nacc = jnp.zeros((16,), _f32)
            for (ub, pb, nb) in ((ua_v, pa_v, na_v), (ub_v, pb_v, nb_v)):
                for j in range(HD // 16):
                    sl = pl.ds(j * 16, 16)
                    gu = ub[e, sl]
                    gp = pb[e, sl]
                    gn = nb[e, sl]
                    pacc = pacc + gu * gp
                    nacc = nacc + gu * gn
                    racc = racc + gu * gu + gp * gp + gn * gn
            lane = ii == m
            pos16 = jnp.where(lane, _hsum16(pacc), pos16)
            neg16 = jnp.where(lane, _hsum16(nacc), neg16)
        pos_v[pl.ds(g * 16, 16)] = pos16 * 0.0625
        neg_v[pl.ds(g * 16, 16)] = neg16 * 0.0625
        return racc

    racc = lax.fori_loop(0, BW // 16, _grp, racc0)
    reg_v[pl.ds(0, 16)] = racc * 0.0625

    pltpu.sync_copy(pos_v, pos_hbm.at[pl.ds(base, BW)])
    pltpu.sync_copy(neg_v, neg_hbm.at[pl.ds(base, BW)])
    pltpu.sync_copy(reg_v, reg_hbm.at[wid])


def _make_final():
    mesh = plsc.VectorSubcoreMesh(core_axis_name="c", subcore_axis_name="s")
    return pl.kernel(
        _final_body,
        out_type=(
            jax.ShapeDtypeStruct((BATCH,), _f32),
            jax.ShapeDtypeStruct((BATCH,), _f32),
            jax.ShapeDtypeStruct((NC * NS, 16), _f32),
        ),
        mesh=mesh,
        compiler_params=pltpu.CompilerParams(use_tc_tiling_on_sc=False),
        scratch_types=[
            pltpu.VMEM((BW,), _i32),
            pltpu.VMEM((BW, HD), _f32),
            pltpu.VMEM((BW, HD), _f32),
            pltpu.VMEM((BW, HD), _f32),
            pltpu.VMEM((BW, HD), _f32),
            pltpu.VMEM((BW, HD), _f32),
            pltpu.VMEM((BW, HD), _f32),
            pltpu.VMEM((BW,), _f32),
            pltpu.VMEM((BW,), _f32),
            pltpu.VMEM((16,), _f32),
            pltpu.SemaphoreType.DMA,
        ],
    )


def _loss_body(pos_ref, neg_ref, reg_ref, out_ref):
    z = pos_ref[...] - neg_ref[...]
    loss = -jnp.mean(jnp.log(jax.nn.sigmoid(z)))
    reg = jnp.sum(reg_ref[...])
    out_ref[...] = jnp.full((1, 1), loss + LMBD * (reg * 0.5) / BATCH, _f32)


def kernel(edge_index, edge_weight, user_table, item_table,
           user_id, item_id, neg_item_id):
    src = edge_index[0].astype(_i32)
    dst = edge_index[1].astype(_i32)
    w = edge_weight.astype(_f32)
    pad = jnp.zeros((NPAD - N_NODES, D), _f32)
    h0full = jnp.concatenate([user_table, item_table, pad], axis=0)
    h0 = jnp.stack([h0full[:, :HD], h0full[:, HD:]])

    accum = _make_accum()
    h1 = _normalize(accum(src, dst, w, h0))
    h2 = _normalize(accum(src, dst, w, h1))
    h3 = _normalize(accum(src, dst, w, h2))

    final = _make_final()
    pos, neg, regp = final(
        h0, h1, h2, h3,
        user_id.astype(_i32),
        (item_id + USER_N).astype(_i32),
        (neg_item_id + USER_N).astype(_i32),
    )

    out = pl.pallas_call(
        _loss_body,
        out_shape=jax.ShapeDtypeStruct((1, 1), _f32),
    )(pos.reshape(32, 128), neg.reshape(32, 128), regp)
    return out[0, 0]
